# Initial kernel scaffold; baseline (speedup 1.0000x reference)
import jax
import jax.numpy as jnp
from jax.experimental import pallas as pl


def kernel(features, domain_points, range_points, adjacency, mlp_w0, mlp_w1, mlp_w2, mlp_w3, mlp_w4, mlp_w5, wm_w0, wm_b0, wm_w1, wm_b1, wm_w2, wm_b2, wm_w3, wm_b3):
    raise NotImplementedError("write your pallas kernel here")


# speedup vs baseline: 2.0821x; 2.0821x over previous
import jax
import jax.numpy as jnp
import numpy as np
from jax.experimental import pallas as pl

N_PTS = 1024
DIM = 2
CIN = 8
COUT = 8
B_SZ = 4
ALPHA = (N_PTS / 16.0) ** 2

I_T = 256
J_T = 128
JG = J_T // 8
P8 = JG * I_T

_PI_HI = np.float32(3.1415927410125732)
_PI_LO = np.float32(-8.742277657347586e-08)
_INV_PI = np.float32(0.3183098861837907)


def _psin(x):
    kf = jnp.round(x * _INV_PI)
    r = (x - kf * _PI_HI) - kf * _PI_LO
    r2 = r * r
    p = np.float32(-2.5052108e-08)
    p = p * r2 + np.float32(2.7557319e-06)
    p = p * r2 + np.float32(-1.9841270e-04)
    p = p * r2 + np.float32(8.3333333e-03)
    p = p * r2 + np.float32(-1.6666667e-01)
    s = r + r * (r2 * p)
    odd = (kf.astype(jnp.int32) & 1) == 1
    return jnp.where(odd, -s, s)


def _dense_body(ftc_ref, rho_ref, x_ref, y128a_ref, y128b_ref, y64a_ref,
                y64b_ref, w0l_ref, bd1, bd2, bd3, bd4, w5_ref, out_ref):
    i = pl.program_id(1)
    x = x_ref[...]

    def brow(v):
        W = v.shape[-1]
        return jnp.broadcast_to(v[:, None, :], (JG, I_T, W)).reshape(P8, W)

    def bcol(v, W):
        return jnp.broadcast_to(v[None, :, :], (JG, I_T, W)).reshape(P8, W)

    def bf(v):
        return v.astype(jnp.bfloat16).astype(jnp.float32)

    d0 = bf(brow(y128a_ref[...]) - bcol(x[:, 0:1], 128))
    d1 = bf(brow(y128b_ref[...]) - bcol(x[:, 1:2], 128))
    h = _psin(d0 * bf(w0l_ref[0:1, :]) + d1 * bf(w0l_ref[1:2, :]))
    for bd in (bd1, bd2, bd3, bd4):
        h = _psin(jnp.dot(h, bd[...], preferred_element_type=jnp.float32))

    e0 = brow(y64a_ref[...]) - bcol(x[:, 0:1], 64)
    e1 = brow(y64b_ref[...]) - bcol(x[:, 1:2], 64)
    r2 = e0 * e0 + e1 * e1
    inside = r2 < (1.0 / ALPHA)
    denom = jnp.where(inside, 1.0 - ALPHA * r2, 1.0)
    bump = jnp.where(inside, jnp.exp(-1.0 / denom), 0.0)

    rho = rho_ref[...]
    acc = [jnp.zeros((JG, 64), jnp.float32) for _ in range(B_SZ)]
    for c in range(CIN):
        F = _psin(jnp.dot(h, w5_ref[c], preferred_element_type=jnp.float32))
        F = F * bump
        F3 = F.reshape(JG, I_T, 64)
        for b in range(B_SZ):
            g = ftc_ref[c * B_SZ + b] * rho
            acc[b] = acc[b] + jnp.sum(F3 * g[None, :, :], axis=1)

    for b in range(B_SZ):
        @pl.when(i == 0)
        def _(b=b):
            out_ref[b] = acc[b]

        @pl.when(i != 0)
        def _(b=b):
            out_ref[b] = out_ref[b] + acc[b]


def _dense_call(ftc, rho_col, domain_points, y128a, y128b, y64a, y64b,
                w0l, bds, w5, interpret=False):
    grid = (N_PTS // J_T, N_PTS // I_T)
    return pl.pallas_call(
        _dense_body,
        grid=grid,
        in_specs=[
            pl.BlockSpec((CIN * B_SZ, I_T, 1), lambda j, i: (0, i, 0)),
            pl.BlockSpec((I_T, 1), lambda j, i: (i, 0)),
            pl.BlockSpec((I_T, DIM), lambda j, i: (i, 0)),
            pl.BlockSpec((JG, 128), lambda j, i: (j, 0)),
            pl.BlockSpec((JG, 128), lambda j, i: (j, 0)),
            pl.BlockSpec((JG, 64), lambda j, i: (j, 0)),
            pl.BlockSpec((JG, 64), lambda j, i: (j, 0)),
            pl.BlockSpec((2, 128), lambda j, i: (0, 0)),
            pl.BlockSpec((128, 128), lambda j, i: (0, 0)),
            pl.BlockSpec((128, 128), lambda j, i: (0, 0)),
            pl.BlockSpec((128, 128), lambda j, i: (0, 0)),
            pl.BlockSpec((128, 128), lambda j, i: (0, 0)),
            pl.BlockSpec((CIN, 128, 64), lambda j, i: (0, 0, 0)),
        ],
        out_specs=pl.BlockSpec((B_SZ, JG, 64), lambda j, i: (0, j, 0)),
        out_shape=jax.ShapeDtypeStruct((B_SZ, N_PTS // 8, 64), jnp.float32),
        interpret=interpret,
    )(ftc, rho_col, domain_points, y128a, y128b, y64a, y64b, w0l, *bds, w5)


def _rho_jnp(points, adjacency, wm_ws, wm_bs):
    el_points = points[adjacency].reshape(-1, 3 * DIM)
    h = el_points
    for W, b in zip(wm_ws, wm_bs):
        h = 1.0 / (1.0 + jnp.exp(-(h @ W + b)))
    w = jnp.zeros((points.shape[0],), dtype=h.dtype)
    w = w.at[adjacency.reshape(-1)].add(h.reshape(-1))
    return w


def kernel(features, domain_points, range_points, adjacency, mlp_w0, mlp_w1, mlp_w2, mlp_w3, mlp_w4, mlp_w5, wm_w0, wm_b0, wm_w1, wm_b1, wm_w2, wm_b2, wm_w3, wm_b3):
    mts = []
    din = DIM
    for W in (mlp_w0, mlp_w1, mlp_w2, mlp_w3, mlp_w4, mlp_w5):
        dout = W.size // din
        mts.append(W.reshape(-1).reshape(dout, din).T)
        din = dout

    eye8 = jnp.eye(8, dtype=jnp.float32)
    w0l = jnp.stack([jnp.tile(mts[0][d], 8) for d in range(DIM)])
    bds = [jnp.kron(eye8, mts[l]) for l in range(1, 5)]
    w5 = jnp.stack([jnp.kron(eye8, mts[5][:, c * COUT:(c + 1) * COUT])
                    for c in range(CIN)])

    y128 = [jnp.repeat(range_points[:, d].reshape(N_PTS // 8, 8), 16, axis=1)
            for d in range(DIM)]
    y64 = [jnp.repeat(range_points[:, d].reshape(N_PTS // 8, 8), 8, axis=1)
           for d in range(DIM)]

    rho = _rho_jnp(domain_points, adjacency, [wm_w0, wm_w1, wm_w2, wm_w3],
                   [wm_b0, wm_b1, wm_b2, wm_b3])
    rho_col = rho.reshape(N_PTS, 1)
    ftc = jnp.transpose(features, (2, 0, 1)).reshape(CIN * B_SZ, N_PTS, 1)

    out = _dense_call(ftc, rho_col, domain_points, y128[0], y128[1],
                      y64[0], y64[1], w0l, bds, w5)
    return out.reshape(B_SZ, N_PTS, COUT)


# speedup vs baseline: 2.1936x; 1.0536x over previous
import functools

import jax
import jax.numpy as jnp
import numpy as np
from jax import lax
from jax.experimental import pallas as pl
from jax.experimental.pallas import tpu as pltpu
from jax.experimental.pallas import tpu_sc as plsc

N_EL = 2048
_WM_DIMS = ((6, 8), (8, 8), (8, 8), (8, 3))
_N_WROWS = sum((din + 1) * dout for din, dout in _WM_DIMS)

N_PTS = 1024
DIM = 2
CIN = 8
COUT = 8
B_SZ = 4
ALPHA = (N_PTS / 16.0) ** 2

I_T = 256
J_T = 128
JG = J_T // 8
P8 = JG * I_T

_PI_HI = np.float32(3.1415927410125732)
_PI_LO = np.float32(-8.742277657347586e-08)
_INV_PI = np.float32(0.3183098861837907)


def _psin(x):
    kf = jnp.round(x * _INV_PI)
    r = (x - kf * _PI_HI) - kf * _PI_LO
    r2 = r * r
    p = np.float32(-2.5052108e-08)
    p = p * r2 + np.float32(2.7557319e-06)
    p = p * r2 + np.float32(-1.9841270e-04)
    p = p * r2 + np.float32(8.3333333e-03)
    p = p * r2 + np.float32(-1.6666667e-01)
    s = r + r * (r2 * p)
    odd = (kf.astype(jnp.int32) & 1) == 1
    return jnp.where(odd, -s, s)


def _dense_body(ftc_ref, rho_ref, x_ref, y128a_ref, y128b_ref, y64a_ref,
                y64b_ref, w0l_ref, bd1, bd2, bd3, bd4, w5_ref, out_ref):
    i = pl.program_id(1)
    x = x_ref[...]

    def brow(v):
        W = v.shape[-1]
        return jnp.broadcast_to(v[:, None, :], (JG, I_T, W)).reshape(P8, W)

    def bcol(v, W):
        return jnp.broadcast_to(v[None, :, :], (JG, I_T, W)).reshape(P8, W)

    def bf(v):
        return v.astype(jnp.bfloat16).astype(jnp.float32)

    d0 = bf(brow(y128a_ref[...]) - bcol(x[:, 0:1], 128))
    d1 = bf(brow(y128b_ref[...]) - bcol(x[:, 1:2], 128))
    h = _psin(d0 * bf(w0l_ref[0:1, :]) + d1 * bf(w0l_ref[1:2, :]))
    for bd in (bd1, bd2, bd3, bd4):
        h = _psin(jnp.dot(h, bd[...], preferred_element_type=jnp.float32))

    e0 = brow(y64a_ref[...]) - bcol(x[:, 0:1], 64)
    e1 = brow(y64b_ref[...]) - bcol(x[:, 1:2], 64)
    r2 = e0 * e0 + e1 * e1
    inside = r2 < (1.0 / ALPHA)
    denom = jnp.where(inside, 1.0 - ALPHA * r2, 1.0)
    bump = jnp.where(inside, jnp.exp(-1.0 / denom), 0.0)

    rho = rho_ref[0] + rho_ref[1]
    acc = [jnp.zeros((JG, 64), jnp.float32) for _ in range(B_SZ)]
    for c in range(CIN):
        F = _psin(jnp.dot(h, w5_ref[c], preferred_element_type=jnp.float32))
        F = F * bump
        F3 = F.reshape(JG, I_T, 64)
        for b in range(B_SZ):
            g = ftc_ref[c * B_SZ + b] * rho
            acc[b] = acc[b] + jnp.sum(F3 * g[None, :, :], axis=1)

    for b in range(B_SZ):
        @pl.when(i == 0)
        def _(b=b):
            out_ref[b] = acc[b]

        @pl.when(i != 0)
        def _(b=b):
            out_ref[b] = out_ref[b] + acc[b]


def _dense_call(ftc, rho_col, domain_points, y128a, y128b, y64a, y64b,
                w0l, bds, w5, interpret=False):
    grid = (N_PTS // J_T, N_PTS // I_T)
    return pl.pallas_call(
        _dense_body,
        grid=grid,
        in_specs=[
            pl.BlockSpec((CIN * B_SZ, I_T, 1), lambda j, i: (0, i, 0)),
            pl.BlockSpec((2, I_T, 1), lambda j, i: (0, i, 0)),
            pl.BlockSpec((I_T, DIM), lambda j, i: (i, 0)),
            pl.BlockSpec((JG, 128), lambda j, i: (j, 0)),
            pl.BlockSpec((JG, 128), lambda j, i: (j, 0)),
            pl.BlockSpec((JG, 64), lambda j, i: (j, 0)),
            pl.BlockSpec((JG, 64), lambda j, i: (j, 0)),
            pl.BlockSpec((2, 128), lambda j, i: (0, 0)),
            pl.BlockSpec((128, 128), lambda j, i: (0, 0)),
            pl.BlockSpec((128, 128), lambda j, i: (0, 0)),
            pl.BlockSpec((128, 128), lambda j, i: (0, 0)),
            pl.BlockSpec((128, 128), lambda j, i: (0, 0)),
            pl.BlockSpec((CIN, 128, 64), lambda j, i: (0, 0, 0)),
        ],
        out_specs=pl.BlockSpec((B_SZ, JG, 64), lambda j, i: (0, j, 0)),
        out_shape=jax.ShapeDtypeStruct((B_SZ, N_PTS // 8, 64), jnp.float32),
        interpret=interpret,
    )(ftc, rho_col, domain_points, y128a, y128b, y64a, y64b, w0l, *bds, w5)


def _rho_sc_body(ptsT, wvec, adj3, out, px_v, py_v, wv, adj_v, hbuf, zbuf, shared):
    cid = lax.axis_index("c")
    sid = lax.axis_index("s")
    wid = cid * 16 + sid

    pltpu.sync_copy(ptsT.at[0], px_v)
    pltpu.sync_copy(ptsT.at[1], py_v)
    pltpu.sync_copy(wvec, wv)
    pltpu.sync_copy(adj3.at[wid], adj_v)

    @pl.when(sid == 0)
    def _():
        for t in range(N_PTS // 16):
            zbuf[pl.ds(t * 16, 16)] = jnp.zeros((16,), jnp.float32)
        pltpu.sync_copy(zbuf, shared)

    plsc.subcore_barrier()

    for g in range(4):
        idx = [adj_v[k, pl.ds(g * 16, 16)] for k in range(3)]
        el = []
        for k in range(3):
            el.append(plsc.load_gather(px_v, [idx[k]]))
            el.append(plsc.load_gather(py_v, [idx[k]]))
        act = el
        row = 0
        for din, dout in _WM_DIMS:
            nxt = []
            for p in range(dout):
                a = wv[row]
                row += 1
                for q in range(din):
                    a = a + act[q] * wv[row]
                    row += 1
                nxt.append(1.0 / (1.0 + jnp.exp(-a)))
            act = nxt
        for k in range(3):
            hbuf[k, pl.ds(g * 16, 16)] = act[k]

    for k in range(3):
        pltpu.sync_copy(hbuf.at[k], shared.at[adj_v.at[k]], add=True)

    plsc.subcore_barrier()

    @pl.when(sid == 0)
    def _():
        pltpu.sync_copy(shared, out.at[cid])


_rho_sc = functools.partial(
    pl.kernel,
    mesh=plsc.VectorSubcoreMesh(core_axis_name="c", subcore_axis_name="s"),
    out_type=jax.ShapeDtypeStruct((2, N_PTS), jnp.float32),
    compiler_params=pltpu.CompilerParams(needs_layout_passes=False),
    scratch_types=[
        pltpu.VMEM((N_PTS,), jnp.float32),
        pltpu.VMEM((N_PTS,), jnp.float32),
        pltpu.VMEM((_N_WROWS, 16), jnp.float32),
        pltpu.VMEM((3, 64), jnp.int32),
        pltpu.VMEM((3, 64), jnp.float32),
        pltpu.VMEM((N_PTS,), jnp.float32),
        pltpu.VMEM_SHARED((N_PTS,), jnp.float32),
    ],
)(_rho_sc_body)


def _rho_weights(wm_ws, wm_bs):
    lays = []
    for W, b in zip(wm_ws, wm_bs):
        lays.append(jnp.concatenate([b[None, :], W], axis=0).T.reshape(-1))
    wvec = jnp.concatenate(lays)
    return jnp.tile(wvec[:, None], (1, 16))


def kernel(features, domain_points, range_points, adjacency, mlp_w0, mlp_w1, mlp_w2, mlp_w3, mlp_w4, mlp_w5, wm_w0, wm_b0, wm_w1, wm_b1, wm_w2, wm_b2, wm_w3, wm_b3):
    mts = []
    din = DIM
    for W in (mlp_w0, mlp_w1, mlp_w2, mlp_w3, mlp_w4, mlp_w5):
        dout = W.size // din
        mts.append(W.reshape(-1).reshape(dout, din).T)
        din = dout

    eye8 = jnp.eye(8, dtype=jnp.float32)
    w0l = jnp.stack([jnp.tile(mts[0][d], 8) for d in range(DIM)])
    bds = [jnp.kron(eye8, mts[l]) for l in range(1, 5)]
    w5 = jnp.stack([jnp.kron(eye8, mts[5][:, c * COUT:(c + 1) * COUT])
                    for c in range(CIN)])

    y128 = [jnp.repeat(range_points[:, d].reshape(N_PTS // 8, 8), 16, axis=1)
            for d in range(DIM)]
    y64 = [jnp.repeat(range_points[:, d].reshape(N_PTS // 8, 8), 8, axis=1)
           for d in range(DIM)]

    ptsT = domain_points.T
    adj3 = adjacency.T.reshape(3, 32, 64).transpose(1, 0, 2)
    wvec16 = _rho_weights([wm_w0, wm_w1, wm_w2, wm_w3],
                          [wm_b0, wm_b1, wm_b2, wm_b3])
    rho2 = _rho_sc(ptsT, wvec16, adj3)
    rho_col = rho2.reshape(2, N_PTS, 1)
    ftc = jnp.transpose(features, (2, 0, 1)).reshape(CIN * B_SZ, N_PTS, 1)

    out = _dense_call(ftc, rho_col, domain_points, y128[0], y128[1],
                      y64[0], y64[1], w0l, bds, w5)
    return out.reshape(B_SZ, N_PTS, COUT)


# speedup vs baseline: 3.0264x; 1.3796x over previous
import functools

import jax
import jax.numpy as jnp
import numpy as np
from jax import lax
from jax.experimental import pallas as pl
from jax.experimental.pallas import tpu as pltpu
from jax.experimental.pallas import tpu_sc as plsc

N_EL = 2048
_WM_DIMS = ((6, 8), (8, 8), (8, 8), (8, 3))
_N_WROWS = sum((din + 1) * dout for din, dout in _WM_DIMS)

N_PTS = 1024
DIM = 2
CIN = 8
COUT = 8
B_SZ = 4
ALPHA = (N_PTS / 16.0) ** 2

I_T = 256
J_T = 128
JG = J_T // 8
P8 = JG * I_T

_PI_HI = np.float32(3.1415927410125732)
_PI_LO = np.float32(-8.742277657347586e-08)
_INV_PI = np.float32(0.3183098861837907)


def _psin(x):
    kf = jnp.round(x * _INV_PI)
    r = (x - kf * _PI_HI) - kf * _PI_LO
    r2 = r * r
    p = np.float32(2.7557319e-06)
    p = p * r2 + np.float32(-1.9841270e-04)
    p = p * r2 + np.float32(8.3333333e-03)
    p = p * r2 + np.float32(-1.6666667e-01)
    s = r + r * (r2 * p)
    odd = (kf.astype(jnp.int32) & 1) == 1
    return jnp.where(odd, -s, s)


def _dense_body(ft_ref, rho_ref, x_ref, y128a_ref, y128b_ref, y64a_ref,
                y64b_ref, w0l_ref, bd1, bd2, bd3, bd4, w5_ref, out_ref):
    i = pl.program_id(1)
    x = x_ref[...]

    def brow(v):
        W = v.shape[-1]
        return jnp.broadcast_to(v[:, None, :], (JG, I_T, W)).reshape(P8, W)

    def bcol(v, W):
        return jnp.broadcast_to(v[None, :, :], (JG, I_T, W)).reshape(P8, W)

    def bf(v):
        return v.astype(jnp.bfloat16).astype(jnp.float32)

    d0 = bf(brow(y128a_ref[...]) - bcol(x[:, 0:1], 128))
    d1 = bf(brow(y128b_ref[...]) - bcol(x[:, 1:2], 128))
    h = _psin(d0 * bf(w0l_ref[0:1, :]) + d1 * bf(w0l_ref[1:2, :]))
    for bd in (bd1, bd2, bd3, bd4):
        h = _psin(jnp.dot(h, bd[...], preferred_element_type=jnp.float32))

    e0 = brow(y64a_ref[...]) - bcol(x[:, 0:1], 64)
    e1 = brow(y64b_ref[...]) - bcol(x[:, 1:2], 64)
    r2 = e0 * e0 + e1 * e1
    inside = r2 < (1.0 / ALPHA)
    denom = jnp.where(inside, 1.0 - ALPHA * r2, 1.0)
    bump = jnp.where(inside, jnp.exp(-1.0 / denom), 0.0)

    rho_row = rho_ref[0:1, :] + rho_ref[1:2, :]
    accs = [jnp.zeros((B_SZ, 64), jnp.float32) for _ in range(JG)]
    for c in range(CIN):
        F = _psin(jnp.dot(h, w5_ref[c], preferred_element_type=jnp.float32))
        F = F * bump
        g = ft_ref[c] * rho_row
        for jg in range(JG):
            Fjg = F[jg * I_T:(jg + 1) * I_T, :]
            accs[jg] = accs[jg] + jnp.dot(g, Fjg, preferred_element_type=jnp.float32)

    for jg in range(JG):
        @pl.when(i == 0)
        def _(jg=jg):
            out_ref[jg] = accs[jg]

        @pl.when(i != 0)
        def _(jg=jg):
            out_ref[jg] = out_ref[jg] + accs[jg]


def _dense_call(ft, rho2, domain_points, y128a, y128b, y64a, y64b,
                w0l, bds, w5, interpret=False):
    grid = (N_PTS // J_T, N_PTS // I_T)
    return pl.pallas_call(
        _dense_body,
        grid=grid,
        in_specs=[
            pl.BlockSpec((CIN, B_SZ, I_T), lambda j, i: (0, 0, i)),
            pl.BlockSpec((2, I_T), lambda j, i: (0, i)),
            pl.BlockSpec((I_T, DIM), lambda j, i: (i, 0)),
            pl.BlockSpec((JG, 128), lambda j, i: (j, 0)),
            pl.BlockSpec((JG, 128), lambda j, i: (j, 0)),
            pl.BlockSpec((JG, 64), lambda j, i: (j, 0)),
            pl.BlockSpec((JG, 64), lambda j, i: (j, 0)),
            pl.BlockSpec((2, 128), lambda j, i: (0, 0)),
            pl.BlockSpec((128, 128), lambda j, i: (0, 0)),
            pl.BlockSpec((128, 128), lambda j, i: (0, 0)),
            pl.BlockSpec((128, 128), lambda j, i: (0, 0)),
            pl.BlockSpec((128, 128), lambda j, i: (0, 0)),
            pl.BlockSpec((CIN, 128, 64), lambda j, i: (0, 0, 0)),
        ],
        out_specs=pl.BlockSpec((JG, B_SZ, 64), lambda j, i: (j, 0, 0)),
        out_shape=jax.ShapeDtypeStruct((N_PTS // 8, B_SZ, 64), jnp.float32),
        interpret=interpret,
    )(ft, rho2, domain_points, y128a, y128b, y64a, y64b, w0l, *bds, w5)


def _rho_sc_body(ptsT, wvec, adj3, out, px_v, py_v, wv, adj_v, hbuf, zbuf, shared):
    cid = lax.axis_index("c")
    sid = lax.axis_index("s")
    wid = cid * 16 + sid

    pltpu.sync_copy(ptsT.at[0], px_v)
    pltpu.sync_copy(ptsT.at[1], py_v)
    pltpu.sync_copy(wvec, wv)
    pltpu.sync_copy(adj3.at[wid], adj_v)

    @pl.when(sid == 0)
    def _():
        for t in range(N_PTS // 16):
            zbuf[pl.ds(t * 16, 16)] = jnp.zeros((16,), jnp.float32)
        pltpu.sync_copy(zbuf, shared)

    plsc.subcore_barrier()

    for g in range(4):
        idx = [adj_v[k, pl.ds(g * 16, 16)] for k in range(3)]
        el = []
        for k in range(3):
            el.append(plsc.load_gather(px_v, [idx[k]]))
            el.append(plsc.load_gather(py_v, [idx[k]]))
        act = el
        row = 0
        for din, dout in _WM_DIMS:
            nxt = []
            for p in range(dout):
                a = wv[row]
                row += 1
                for q in range(din):
                    a = a + act[q] * wv[row]
                    row += 1
                nxt.append(1.0 / (1.0 + jnp.exp(-a)))
            act = nxt
        for k in range(3):
            hbuf[k, pl.ds(g * 16, 16)] = act[k]

    for k in range(3):
        pltpu.sync_copy(hbuf.at[k], shared.at[adj_v.at[k]], add=True)

    plsc.subcore_barrier()

    @pl.when(sid == 0)
    def _():
        pltpu.sync_copy(shared, out.at[cid])


_rho_sc = functools.partial(
    pl.kernel,
    mesh=plsc.VectorSubcoreMesh(core_axis_name="c", subcore_axis_name="s"),
    out_type=jax.ShapeDtypeStruct((2, N_PTS), jnp.float32),
    compiler_params=pltpu.CompilerParams(needs_layout_passes=False),
    scratch_types=[
        pltpu.VMEM((N_PTS,), jnp.float32),
        pltpu.VMEM((N_PTS,), jnp.float32),
        pltpu.VMEM((_N_WROWS, 16), jnp.float32),
        pltpu.VMEM((3, 64), jnp.int32),
        pltpu.VMEM((3, 64), jnp.float32),
        pltpu.VMEM((N_PTS,), jnp.float32),
        pltpu.VMEM_SHARED((N_PTS,), jnp.float32),
    ],
)(_rho_sc_body)


def _rho_weights(wm_ws, wm_bs):
    lays = []
    for W, b in zip(wm_ws, wm_bs):
        lays.append(jnp.concatenate([b[None, :], W], axis=0).T.reshape(-1))
    wvec = jnp.concatenate(lays)
    return jnp.tile(wvec[:, None], (1, 16))


def kernel(features, domain_points, range_points, adjacency, mlp_w0, mlp_w1, mlp_w2, mlp_w3, mlp_w4, mlp_w5, wm_w0, wm_b0, wm_w1, wm_b1, wm_w2, wm_b2, wm_w3, wm_b3):
    mts = []
    din = DIM
    for W in (mlp_w0, mlp_w1, mlp_w2, mlp_w3, mlp_w4, mlp_w5):
        dout = W.size // din
        mts.append(W.reshape(-1).reshape(dout, din).T)
        din = dout

    eye8 = jnp.eye(8, dtype=jnp.float32)
    w0l = jnp.stack([jnp.tile(mts[0][d], 8) for d in range(DIM)])
    bds = [jnp.kron(eye8, mts[l]) for l in range(1, 5)]
    w5 = jnp.stack([jnp.kron(eye8, mts[5][:, c * COUT:(c + 1) * COUT])
                    for c in range(CIN)])

    y128 = [jnp.repeat(range_points[:, d].reshape(N_PTS // 8, 8), 16, axis=1)
            for d in range(DIM)]
    y64 = [jnp.repeat(range_points[:, d].reshape(N_PTS // 8, 8), 8, axis=1)
           for d in range(DIM)]

    ptsT = domain_points.T
    adj3 = adjacency.T.reshape(3, 32, 64).transpose(1, 0, 2)
    wvec16 = _rho_weights([wm_w0, wm_w1, wm_w2, wm_w3],
                          [wm_b0, wm_b1, wm_b2, wm_b3])
    rho2 = _rho_sc(ptsT, wvec16, adj3)
    ft = jnp.transpose(features, (2, 0, 1))

    out = _dense_call(ft, rho2, domain_points, y128[0], y128[1],
                      y64[0], y64[1], w0l, bds, w5)
    return jnp.transpose(out, (1, 0, 2)).reshape(B_SZ, N_PTS, COUT)


# speedup vs baseline: 3.2763x; 1.0826x over previous
import functools

import jax
import jax.numpy as jnp
import numpy as np
from jax import lax
from jax.experimental import pallas as pl
from jax.experimental.pallas import tpu as pltpu
from jax.experimental.pallas import tpu_sc as plsc

N_EL = 2048
_WM_DIMS = ((6, 8), (8, 8), (8, 8), (8, 3))
_N_WROWS = sum((din + 1) * dout for din, dout in _WM_DIMS)

N_PTS = 1024
DIM = 2
CIN = 8
COUT = 8
B_SZ = 4
ALPHA = (N_PTS / 16.0) ** 2

I_T = 256
J_T = 128
JG = J_T // 8
P8 = JG * I_T

_PI_HI = np.float32(3.1415927410125732)
_PI_LO = np.float32(-8.742277657347586e-08)
_INV_PI = np.float32(0.3183098861837907)


def _psin(x):
    kf = jnp.round(x * _INV_PI)
    r = x - kf * _PI_HI
    r2 = r * r
    p = np.float32(2.7557319e-06)
    p = p * r2 + np.float32(-1.9841270e-04)
    p = p * r2 + np.float32(8.3333333e-03)
    p = p * r2 + np.float32(-1.6666667e-01)
    s = r + r * (r2 * p)
    odd = (kf.astype(jnp.int32) & 1) == 1
    return jnp.where(odd, -s, s)


def _dense_body(ft_ref, rho_ref, x_ref, y128a_ref, y128b_ref, y64a_ref,
                y64b_ref, w0l_ref, bd1, bd2, bd3, bd4, w5_ref, out_ref):
    i = pl.program_id(1)
    x = x_ref[...]

    def brow(v):
        W = v.shape[-1]
        return jnp.broadcast_to(v[:, None, :], (JG, I_T, W)).reshape(P8, W)

    def bcol(v, W):
        return jnp.broadcast_to(v[None, :, :], (JG, I_T, W)).reshape(P8, W)

    def bf(v):
        return v.astype(jnp.bfloat16).astype(jnp.float32)

    d0 = bf(brow(y128a_ref[...]) - bcol(x[:, 0:1], 128))
    d1 = bf(brow(y128b_ref[...]) - bcol(x[:, 1:2], 128))
    h = _psin(d0 * bf(w0l_ref[0:1, :]) + d1 * bf(w0l_ref[1:2, :]))
    for bd in (bd1, bd2, bd3, bd4):
        h = _psin(jnp.dot(h, bd[...], preferred_element_type=jnp.float32))

    e0 = brow(y64a_ref[...]) - bcol(x[:, 0:1], 64)
    e1 = brow(y64b_ref[...]) - bcol(x[:, 1:2], 64)
    r2 = e0 * e0 + e1 * e1
    inside = r2 < (1.0 / ALPHA)
    denom = jnp.where(inside, 1.0 - ALPHA * r2, 1.0)
    bump = jnp.where(inside, jnp.exp(-1.0 / denom), 0.0)

    rho_row = rho_ref[0:1, :] + rho_ref[1:2, :]
    accs = [jnp.zeros((B_SZ, 64), jnp.float32) for _ in range(JG)]
    for c in range(CIN):
        F = _psin(jnp.dot(h, w5_ref[c], preferred_element_type=jnp.float32))
        F = F * bump
        g = ft_ref[c] * rho_row
        for jg in range(JG):
            Fjg = F[jg * I_T:(jg + 1) * I_T, :]
            accs[jg] = accs[jg] + jnp.dot(g, Fjg, preferred_element_type=jnp.float32)

    for jg in range(JG):
        @pl.when(i == 0)
        def _(jg=jg):
            out_ref[jg] = accs[jg]

        @pl.when(i != 0)
        def _(jg=jg):
            out_ref[jg] = out_ref[jg] + accs[jg]


def _dense_call(ft, rho2, domain_points, y128a, y128b, y64a, y64b,
                w0l, bds, w5, interpret=False):
    grid = (N_PTS // J_T, N_PTS // I_T)
    return pl.pallas_call(
        _dense_body,
        grid=grid,
        in_specs=[
            pl.BlockSpec((CIN, B_SZ, I_T), lambda j, i: (0, 0, i)),
            pl.BlockSpec((2, I_T), lambda j, i: (0, i)),
            pl.BlockSpec((I_T, DIM), lambda j, i: (i, 0)),
            pl.BlockSpec((JG, 128), lambda j, i: (j, 0)),
            pl.BlockSpec((JG, 128), lambda j, i: (j, 0)),
            pl.BlockSpec((JG, 64), lambda j, i: (j, 0)),
            pl.BlockSpec((JG, 64), lambda j, i: (j, 0)),
            pl.BlockSpec((2, 128), lambda j, i: (0, 0)),
            pl.BlockSpec((128, 128), lambda j, i: (0, 0)),
            pl.BlockSpec((128, 128), lambda j, i: (0, 0)),
            pl.BlockSpec((128, 128), lambda j, i: (0, 0)),
            pl.BlockSpec((128, 128), lambda j, i: (0, 0)),
            pl.BlockSpec((CIN, 128, 64), lambda j, i: (0, 0, 0)),
        ],
        out_specs=pl.BlockSpec((JG, B_SZ, 64), lambda j, i: (j, 0, 0)),
        out_shape=jax.ShapeDtypeStruct((N_PTS // 8, B_SZ, 64), jnp.float32),
        interpret=interpret,
    )(ft, rho2, domain_points, y128a, y128b, y64a, y64b, w0l, *bds, w5)


def _rho_sc_body(ptsT, wvec, adj3, out, px_v, py_v, wv, adj_v, hbuf, zbuf, shared):
    cid = lax.axis_index("c")
    sid = lax.axis_index("s")
    wid = cid * 16 + sid

    pltpu.sync_copy(ptsT.at[0], px_v)
    pltpu.sync_copy(ptsT.at[1], py_v)
    pltpu.sync_copy(wvec, wv)
    pltpu.sync_copy(adj3.at[wid], adj_v)

    @pl.when(sid == 0)
    def _():
        for t in range(N_PTS // 16):
            zbuf[pl.ds(t * 16, 16)] = jnp.zeros((16,), jnp.float32)
        pltpu.sync_copy(zbuf, shared)

    plsc.subcore_barrier()

    for g in range(4):
        idx = [adj_v[k, pl.ds(g * 16, 16)] for k in range(3)]
        el = []
        for k in range(3):
            el.append(plsc.load_gather(px_v, [idx[k]]))
            el.append(plsc.load_gather(py_v, [idx[k]]))
        act = el
        row = 0
        for din, dout in _WM_DIMS:
            nxt = []
            for p in range(dout):
                a = wv[row]
                row += 1
                for q in range(din):
                    a = a + act[q] * wv[row]
                    row += 1
                nxt.append(1.0 / (1.0 + jnp.exp(-a)))
            act = nxt
        for k in range(3):
            hbuf[k, pl.ds(g * 16, 16)] = act[k]

    for k in range(3):
        pltpu.sync_copy(hbuf.at[k], shared.at[adj_v.at[k]], add=True)

    plsc.subcore_barrier()

    @pl.when(sid == 0)
    def _():
        pltpu.sync_copy(shared, out.at[cid])


_rho_sc = functools.partial(
    pl.kernel,
    mesh=plsc.VectorSubcoreMesh(core_axis_name="c", subcore_axis_name="s"),
    out_type=jax.ShapeDtypeStruct((2, N_PTS), jnp.float32),
    compiler_params=pltpu.CompilerParams(needs_layout_passes=False),
    scratch_types=[
        pltpu.VMEM((N_PTS,), jnp.float32),
        pltpu.VMEM((N_PTS,), jnp.float32),
        pltpu.VMEM((_N_WROWS, 16), jnp.float32),
        pltpu.VMEM((3, 64), jnp.int32),
        pltpu.VMEM((3, 64), jnp.float32),
        pltpu.VMEM((N_PTS,), jnp.float32),
        pltpu.VMEM_SHARED((N_PTS,), jnp.float32),
    ],
)(_rho_sc_body)


def _rho_weights(wm_ws, wm_bs):
    lays = []
    for W, b in zip(wm_ws, wm_bs):
        lays.append(jnp.concatenate([b[None, :], W], axis=0).T.reshape(-1))
    wvec = jnp.concatenate(lays)
    return jnp.tile(wvec[:, None], (1, 16))


def kernel(features, domain_points, range_points, adjacency, mlp_w0, mlp_w1, mlp_w2, mlp_w3, mlp_w4, mlp_w5, wm_w0, wm_b0, wm_w1, wm_b1, wm_w2, wm_b2, wm_w3, wm_b3):
    mts = []
    din = DIM
    for W in (mlp_w0, mlp_w1, mlp_w2, mlp_w3, mlp_w4, mlp_w5):
        dout = W.size // din
        mts.append(W.reshape(-1).reshape(dout, din).T)
        din = dout

    eye8 = jnp.eye(8, dtype=jnp.float32)
    w0l = jnp.stack([jnp.tile(mts[0][d], 8) for d in range(DIM)])
    bds = [jnp.kron(eye8, mts[l]) for l in range(1, 5)]
    w5 = jnp.stack([jnp.kron(eye8, mts[5][:, c * COUT:(c + 1) * COUT])
                    for c in range(CIN)])

    y128 = [jnp.repeat(range_points[:, d].reshape(N_PTS // 8, 8), 16, axis=1)
            for d in range(DIM)]
    y64 = [jnp.repeat(range_points[:, d].reshape(N_PTS // 8, 8), 8, axis=1)
           for d in range(DIM)]

    ptsT = domain_points.T
    adj3 = adjacency.T.reshape(3, 32, 64).transpose(1, 0, 2)
    wvec16 = _rho_weights([wm_w0, wm_w1, wm_w2, wm_w3],
                          [wm_b0, wm_b1, wm_b2, wm_b3])
    rho2 = _rho_sc(ptsT, wvec16, adj3)
    ft = jnp.transpose(features, (2, 0, 1))

    out = _dense_call(ft, rho2, domain_points, y128[0], y128[1],
                      y64[0], y64[1], w0l, bds, w5)
    return jnp.transpose(out, (1, 0, 2)).reshape(B_SZ, N_PTS, COUT)
